# baseline (speedup 1.0000x reference)
import functools

import jax
import jax.numpy as jnp
from jax import lax
from jax.experimental import pallas as pl
from jax.experimental.pallas import tpu as pltpu
from jax.experimental.pallas import tpu_sc as plsc

BATCH = 4
MAX_LEN = 4096
D_MODEL = 1024
NC = 2
NS = 16
LANES = 16
NW = NC * NS
ROWS_PER_W = MAX_LEN // NW
CHUNK = 16
NCHUNK = ROWS_PER_W // CHUNK
CHUNK_ELEMS = CHUNK * D_MODEL
VREGS = CHUNK_ELEMS // LANES


def _sc_add(x_flat, emb_flat):
    mesh = plsc.VectorSubcoreMesh(core_axis_name="c", subcore_axis_name="s")

    @functools.partial(
        pl.kernel,
        out_type=jax.ShapeDtypeStruct((BATCH, MAX_LEN * D_MODEL), jnp.float32),
        mesh=mesh,
        scratch_types=[
            pltpu.VMEM((CHUNK_ELEMS,), jnp.float32),
            pltpu.VMEM((CHUNK_ELEMS,), jnp.float32),
        ],
    )
    def body(x_hbm, emb_hbm, out_hbm, ebuf, xbuf):
        wid = lax.axis_index("s") * NC + lax.axis_index("c")
        base = wid * ROWS_PER_W * D_MODEL

        def do_chunk(ci, carry):
            off = base + ci * CHUNK_ELEMS
            pltpu.sync_copy(emb_hbm.at[pl.ds(off, CHUNK_ELEMS)], ebuf)
            for b in range(BATCH):
                pltpu.sync_copy(x_hbm.at[b, pl.ds(off, CHUNK_ELEMS)], xbuf)

                def add_one(i, c):
                    s = pl.ds(i * LANES, LANES)
                    plsc.addupdate(xbuf.at[s], ebuf[s])
                    return c

                lax.fori_loop(0, VREGS, add_one, 0)
                pltpu.sync_copy(xbuf, out_hbm.at[b, pl.ds(off, CHUNK_ELEMS)])
            return carry

        lax.fori_loop(0, NCHUNK, do_chunk, 0)

    return body(x_flat, emb_flat)


def kernel(x, emb):
    x_flat = x.reshape(BATCH, MAX_LEN * D_MODEL)
    emb_flat = emb.reshape(MAX_LEN * D_MODEL)
    out = _sc_add(x_flat, emb_flat)
    return out.reshape(BATCH, MAX_LEN, D_MODEL)


# speedup vs baseline: 1.7218x; 1.7218x over previous
import functools

import jax
import jax.numpy as jnp
from jax import lax
from jax.experimental import pallas as pl
from jax.experimental.pallas import tpu as pltpu
from jax.experimental.pallas import tpu_sc as plsc

BATCH = 4
MAX_LEN = 4096
D_MODEL = 1024
NC = 2
NS = 16
LANES = 16
NW = NC * NS
ROWS_PER_W = MAX_LEN // NW
CHUNK = 8
NCHUNK = ROWS_PER_W // CHUNK
CHUNK_ELEMS = CHUNK * D_MODEL
VREGS = CHUNK_ELEMS // LANES


def _sc_add(x_flat, emb_flat):
    mesh = plsc.VectorSubcoreMesh(core_axis_name="c", subcore_axis_name="s")

    scratch = (
        [pltpu.VMEM((CHUNK_ELEMS,), jnp.float32) for _ in range(2)]
        + [pltpu.VMEM((CHUNK_ELEMS,), jnp.float32) for _ in range(8)]
        + [pltpu.SemaphoreType.DMA for _ in range(2)]
        + [pltpu.SemaphoreType.DMA for _ in range(2)]
    )

    @functools.partial(
        pl.kernel,
        out_type=jax.ShapeDtypeStruct((BATCH, MAX_LEN * D_MODEL), jnp.float32),
        mesh=mesh,
        scratch_types=scratch,
    )
    def body(x_hbm, emb_hbm, out_hbm, *refs):
        ebuf = refs[0:2]
        xbuf = [refs[2 + s * BATCH:2 + (s + 1) * BATCH] for s in range(2)]
        lsem = refs[10:12]
        ssem = refs[12:14]

        wid = lax.axis_index("s") * NC + lax.axis_index("c")
        base = wid * ROWS_PER_W * D_MODEL

        def issue_loads(ci, st):
            off = base + ci * CHUNK_ELEMS
            descs = [pltpu.async_copy(
                emb_hbm.at[pl.ds(off, CHUNK_ELEMS)], ebuf[st], lsem[st])]
            for b in range(BATCH):
                descs.append(pltpu.async_copy(
                    x_hbm.at[b, pl.ds(off, CHUNK_ELEMS)], xbuf[st][b],
                    lsem[st]))
            return descs

        def issue_stores(ci, st):
            off = base + ci * CHUNK_ELEMS
            return [pltpu.async_copy(
                xbuf[st][b], out_hbm.at[b, pl.ds(off, CHUNK_ELEMS)], ssem[st])
                for b in range(BATCH)]

        load_descs = [None, None]
        store_descs = [None, None]
        load_descs[0] = issue_loads(0, 0)

        for ci in range(NCHUNK):
            cur = ci % 2
            nxt = (ci + 1) % 2
            if ci + 1 < NCHUNK:
                if store_descs[nxt] is not None:
                    for d in store_descs[nxt]:
                        d.wait()
                load_descs[nxt] = issue_loads(ci + 1, nxt)
            for d in load_descs[cur]:
                d.wait()
            for b in range(BATCH):
                xb = xbuf[cur][b]
                eb = ebuf[cur]

                def add_one(i):
                    s = pl.ds(i * LANES, LANES)
                    plsc.addupdate(xb.at[s], eb[s])

                plsc.parallel_loop(0, VREGS, 1, unroll=8)(add_one)
            store_descs[cur] = issue_stores(ci, cur)

        for st in range(2):
            if store_descs[st] is not None:
                for d in store_descs[st]:
                    d.wait()

    return body(x_flat, emb_flat)


def kernel(x, emb):
    x_flat = x.reshape(BATCH, MAX_LEN * D_MODEL)
    emb_flat = emb.reshape(MAX_LEN * D_MODEL)
    out = _sc_add(x_flat, emb_flat)
    return out.reshape(BATCH, MAX_LEN, D_MODEL)


# speedup vs baseline: 3.9822x; 2.3128x over previous
import functools

import jax
import jax.numpy as jnp
from jax import lax
from jax.experimental import pallas as pl
from jax.experimental.pallas import tpu as pltpu
from jax.experimental.pallas import tpu_sc as plsc

BATCH = 4
MAX_LEN = 4096
D_MODEL = 1024
NC = 2
NS = 16
LANES = 16
NW = NC * NS
ROWS_PER_W = MAX_LEN // NW
CHUNK = 8
NCHUNK = ROWS_PER_W // CHUNK
ROW_VREGS = D_MODEL // LANES


def _sc_add(x, emb):
    mesh = plsc.VectorSubcoreMesh(core_axis_name="c", subcore_axis_name="s")

    scratch = (
        [pltpu.VMEM((CHUNK, D_MODEL), jnp.float32) for _ in range(2)]
        + [pltpu.VMEM((CHUNK, D_MODEL), jnp.float32) for _ in range(8)]
        + [pltpu.SemaphoreType.DMA for _ in range(2)]
        + [pltpu.SemaphoreType.DMA for _ in range(2)]
    )

    @functools.partial(
        pl.kernel,
        out_type=jax.ShapeDtypeStruct((BATCH, MAX_LEN, D_MODEL), jnp.float32),
        mesh=mesh,
        scratch_types=scratch,
    )
    def body(x_hbm, emb_hbm, out_hbm, *refs):
        ebuf = refs[0:2]
        xbuf = [refs[2 + s * BATCH:2 + (s + 1) * BATCH] for s in range(2)]
        lsem = refs[10:12]
        ssem = refs[12:14]

        wid = lax.axis_index("s") * NC + lax.axis_index("c")
        base = wid * ROWS_PER_W

        def issue_loads(ci, st):
            r0 = base + ci * CHUNK
            descs = [pltpu.async_copy(
                emb_hbm.at[pl.ds(r0, CHUNK)], ebuf[st], lsem[st])]
            for b in range(BATCH):
                descs.append(pltpu.async_copy(
                    x_hbm.at[b, pl.ds(r0, CHUNK)], xbuf[st][b], lsem[st]))
            return descs

        def issue_stores(ci, st):
            r0 = base + ci * CHUNK
            return [pltpu.async_copy(
                xbuf[st][b], out_hbm.at[b, pl.ds(r0, CHUNK)], ssem[st])
                for b in range(BATCH)]

        load_descs = [None, None]
        store_descs = [None, None]
        load_descs[0] = issue_loads(0, 0)

        for ci in range(NCHUNK):
            cur = ci % 2
            nxt = (ci + 1) % 2
            if ci + 1 < NCHUNK:
                if store_descs[nxt] is not None:
                    for d in store_descs[nxt]:
                        d.wait()
                load_descs[nxt] = issue_loads(ci + 1, nxt)
            for d in load_descs[cur]:
                d.wait()
            for b in range(BATCH):
                xb = xbuf[cur][b]
                eb = ebuf[cur]

                def add_one(i, _xb=xb, _eb=eb):
                    r = lax.shift_right_logical(i, 6)
                    j = lax.bitwise_and(i, ROW_VREGS - 1)
                    s = pl.ds(j * LANES, LANES)
                    plsc.addupdate(_xb.at[r, s], _eb[r, s])

                plsc.parallel_loop(0, CHUNK * ROW_VREGS, 1, unroll=8)(add_one)
            store_descs[cur] = issue_stores(ci, cur)

        for st in range(2):
            if store_descs[st] is not None:
                for d in store_descs[st]:
                    d.wait()

    return body(x, emb)


def kernel(x, emb):
    return _sc_add(x, emb)
